# baseline (speedup 1.0000x reference)
import functools

import jax
import jax.numpy as jnp
import numpy as np
from jax import lax
from jax.experimental import pallas as pl
from jax.experimental.pallas import tpu as pltpu
from jax.experimental.pallas import tpu_sc as plsc

KW = float(np.sqrt(3.0 * 3.0 / np.pi))
NS = 64
C = 400
GROUPS = C // 16
INV63 = np.float32(1.0 / 63.0)


def _sc_projection(blocks, img_flat, strides, dims, n_chunks, Mp):
    T = Mp // 32
    info = plsc.get_sparse_core_info()
    NC = info.num_cores

    mesh = plsc.VectorSubcoreMesh(core_axis_name="c", subcore_axis_name="s")

    @functools.partial(
        pl.kernel,
        mesh=mesh,
        out_type=jax.ShapeDtypeStruct((3 * Mp,), jnp.float32),
        scratch_types=[
            pltpu.VMEM((7 * C,), jnp.float32),
            pltpu.VMEM((7 * C,), jnp.float32),
            pltpu.VMEM((C * NS,), jnp.int32),
            pltpu.VMEM((C * NS,), jnp.int32),
            pltpu.VMEM((C * NS,), jnp.float32),
            pltpu.VMEM((C * NS,), jnp.float32),
            pltpu.VMEM((C,), jnp.float32),
            pltpu.VMEM((C,), jnp.float32),
            pltpu.SemaphoreType.DMA,
            pltpu.SemaphoreType.DMA,
        ],
    )
    def proj_kernel(blk_hbm, img_hbm, out_hbm,
                    qa, qb, ia_, ib_, va, vb, oa, ob, sema, semb):
        wid = lax.axis_index("s") * NC + lax.axis_index("c")

        def load_idx(s, ch, qv, idxv, fmax):
            sa, sb, sc_ = strides[s]
            blk_off = ((s * 32 + wid) * n_chunks + ch) * (7 * C)
            pltpu.sync_copy(blk_hbm.at[pl.ds(blk_off, 7 * C)], qv)
            fa, fb, fc = fmax

            def idx_body(g, _):
                q0a = qv[pl.ds(0 * C + g * 16, 16)]
                q0b = qv[pl.ds(1 * C + g * 16, 16)]
                q0c = qv[pl.ds(2 * C + g * 16, 16)]
                dda = qv[pl.ds(3 * C + g * 16, 16)]
                ddb = qv[pl.ds(4 * C + g * 16, 16)]
                ddc = qv[pl.ds(5 * C + g * 16, 16)]
                base = g * (16 * NS)

                def kk_body(kk, _):
                    for k8 in range(8):
                        k = kk * 8 + k8
                        t = k.astype(jnp.float32) * INV63
                        ua = jnp.clip(q0a + dda * t, 0.0, fa)
                        ub = jnp.clip(q0b + ddb * t, 0.0, fb)
                        uc = jnp.clip(q0c + ddc * t, 0.0, fc)
                        flat = (ua.astype(jnp.int32) * sa
                                + ub.astype(jnp.int32) * sb
                                + uc.astype(jnp.int32) * sc_)
                        idxv[pl.ds(base + k * 16, 16)] = flat
                    return 0

                lax.fori_loop(0, 8, kk_body, 0)
                return 0

            lax.fori_loop(0, GROUPS, idx_body, 0)

        def reduce_store(s, ch, qv, valv, outv):
            def red_body(g, _):
                base = g * (16 * NS)
                acc0 = valv[pl.ds(base + 0 * 16, 16)]
                acc1 = valv[pl.ds(base + 1 * 16, 16)]
                acc2 = valv[pl.ds(base + 2 * 16, 16)]
                acc3 = valv[pl.ds(base + 3 * 16, 16)]
                for k in range(4, NS, 4):
                    acc0 = acc0 + valv[pl.ds(base + k * 16, 16)]
                    acc1 = acc1 + valv[pl.ds(base + (k + 1) * 16, 16)]
                    acc2 = acc2 + valv[pl.ds(base + (k + 2) * 16, 16)]
                    acc3 = acc3 + valv[pl.ds(base + (k + 3) * 16, 16)]
                total = (acc0 + acc1) + (acc2 + acc3)
                outv[pl.ds(g * 16, 16)] = (
                    total * qv[pl.ds(6 * C + g * 16, 16)])
                return 0

            lax.fori_loop(0, GROUPS, red_body, 0)
            pltpu.sync_copy(outv,
                            out_hbm.at[pl.ds(s * Mp + wid * T + ch * C, C)])

        for s in range(3):
            fmax = (np.float32(dims[s][0] - 1), np.float32(dims[s][1] - 1),
                    np.float32(dims[s][2] - 1))

            def pair_body(i, _, s=s, fmax=fmax):
                ch0 = 2 * i
                ch1 = 2 * i + 1
                load_idx(s, ch0, qa, ia_, fmax)
                cpa = pltpu.async_copy(img_hbm.at[ia_], va, sema)
                load_idx(s, ch1, qb, ib_, fmax)
                cpb = pltpu.async_copy(img_hbm.at[ib_], vb, semb)
                cpa.wait()
                reduce_store(s, ch0, qa, va, oa)
                cpb.wait()
                reduce_store(s, ch1, qb, vb, ob)
                return 0

            lax.fori_loop(0, n_chunks // 2, pair_body, 0)

    return proj_kernel(blocks, img_flat)


def kernel(image, grid, center, size, xlors, ylors, zlors):
    X, Y, Z = image.shape
    sx, sy, sz = Y * Z, Z, 1
    perms = ((2, 0, 1), (1, 0, 2), (0, 1, 2))
    strides = ((sz, sx, sy), (sy, sx, sz), (sx, sy, sz))
    dims_all = (X, Y, Z)
    dims = tuple(tuple(dims_all[p] for p in perm) for perm in perms)

    lors_sets = (xlors, ylors, zlors)
    M = max(l.shape[0] for l in lors_sets)
    n_chunks = -(-M // (32 * C))
    if n_chunks % 2:
        n_chunks += 1
    Mp = 32 * C * n_chunks

    voxel = size / grid
    origin = center - size / 2.0

    blks = []
    for lors, perm in zip(lors_sets, perms):
        p0 = lors[:, 0:3]
        d = lors[:, 3:6] - p0
        scale = jnp.sqrt(jnp.sum(d * d, axis=-1)) * (KW / NS)
        o = jnp.stack([origin[perm[0]], origin[perm[1]], origin[perm[2]]])
        v = jnp.stack([voxel[perm[0]], voxel[perm[1]], voxel[perm[2]]])
        q0 = (p0 - o) / v
        dd = d / v
        comp = jnp.concatenate([q0, dd, scale[:, None]], axis=1)
        pad = Mp - lors.shape[0]
        comp = jnp.pad(comp, ((0, pad), (0, 0)))
        blks.append(jnp.transpose(
            comp.reshape(32, n_chunks, C, 7), (0, 1, 3, 2)))

    blocks = jnp.stack(blks).reshape(-1)
    img_flat = image.reshape(-1)

    out = _sc_projection(blocks, img_flat, strides, dims, n_chunks, Mp)
    return (out[0 * Mp:0 * Mp + xlors.shape[0]],
            out[1 * Mp:1 * Mp + ylors.shape[0]],
            out[2 * Mp:2 * Mp + zlors.shape[0]])
